# baseline (speedup 1.0000x reference)
import jax
import jax.numpy as jnp
from jax.experimental import pallas as pl
from jax.experimental.pallas import tpu as pltpu

_NUM_EMB = 1024
_DIM = 64
_B = 16
_HW = 1024
_N_TOK = _B * _HW
_TPB = 1024
_NBLK = _N_TOK // _TPB
_EPS = 1e-12
_COMMIT = 0.25


def _rowsum64(a):
    t = a[:, 0:8]
    for i in range(1, 8):
        t = t + a[:, 8 * i:8 * i + 8]
    u = t[:, 0:4] + t[:, 4:8]
    v = u[:, 0:2] + u[:, 2:4]
    return v[:, 0:1] + v[:, 1:2]


def _vq_body(x_ref, xn_ref, xn2_ref, w_ref, enc_ref, q_ref, loss_ref,
             perp_ref, cnt_acc, loss_acc, wnt_s, wn2_s):
    b = pl.program_id(0)
    x = x_ref[0]
    xn = xn_ref[0]
    xn2 = xn2_ref[0]
    w = w_ref[...]

    @pl.when(b == 0)
    def _prep():
        wn = w / jnp.maximum(jnp.sqrt(_rowsum64(w * w)), _EPS)
        wnt_s[...] = -2.0 * wn.T
        wn2_s[...] = _rowsum64(wn * wn).T

    m2 = jax.lax.dot_general(
        xn, wnt_s[...], (((1,), (0,)), ((), ())),
        preferred_element_type=jnp.float32)
    d = (xn2 + wn2_s[...]) + m2

    dmin = jnp.min(d, axis=1, keepdims=True)
    iota_j = jax.lax.broadcasted_iota(jnp.int32, d.shape, 1)
    idx = jnp.min(jnp.where(d == dmin, iota_j, _NUM_EMB), axis=1)

    enc = (iota_j == idx[:, None]).astype(jnp.float32)
    enc_ref[...] = enc

    q = jax.lax.dot_general(
        enc, w, (((1,), (0,)), ((), ())),
        preferred_element_type=jnp.float32)
    q_ref[0] = x + (q - x)

    blk_sq = jnp.sum((q - x) ** 2)
    ones_row = jnp.full((8, x.shape[0]), 1.0, jnp.float32)
    blk_cnt = jax.lax.dot_general(
        ones_row, enc, (((1,), (0,)), ((), ())),
        preferred_element_type=jnp.float32)[0:1]

    @pl.when(b == 0)
    def _init():
        loss_acc[0, 0] = blk_sq
        cnt_acc[...] = blk_cnt

    @pl.when(b != 0)
    def _accum():
        loss_acc[0, 0] += blk_sq
        cnt_acc[...] += blk_cnt

    @pl.when(b == _NBLK - 1)
    def _finish():
        loss_ref[0, 0] = _COMMIT * (loss_acc[0, 0] / (_N_TOK * _DIM))
        avg = cnt_acc[...] * (1.0 / _N_TOK)
        perp_ref[0, 0] = jnp.exp(-jnp.sum(avg * jnp.log(avg + 1e-10)))


def kernel(inputs, weight, ema_w, ema_cluster_size):
    del ema_w, ema_cluster_size
    x4 = jnp.transpose(inputs, (0, 2, 3, 1))
    flat_x = x4.reshape(-1, _DIM)
    n = jnp.linalg.norm(flat_x, axis=1, keepdims=True)
    flat_x_n = flat_x / jnp.maximum(n, _EPS)
    xn2 = jnp.sum(flat_x_n ** 2, axis=1, keepdims=True)

    x_b = flat_x.reshape(_NBLK, _TPB, _DIM)
    xn_b = flat_x_n.reshape(_NBLK, _TPB, _DIM)
    xn2_b = xn2.reshape(_NBLK, _TPB, 1)

    enc, q_t, loss, perp = pl.pallas_call(
        _vq_body,
        grid=(_NBLK,),
        in_specs=[
            pl.BlockSpec((1, _TPB, _DIM), lambda b: (b, 0, 0)),
            pl.BlockSpec((1, _TPB, _DIM), lambda b: (b, 0, 0)),
            pl.BlockSpec((1, _TPB, 1), lambda b: (b, 0, 0)),
            pl.BlockSpec((_NUM_EMB, _DIM), lambda b: (0, 0)),
        ],
        out_specs=[
            pl.BlockSpec((_TPB, _NUM_EMB), lambda b: (b, 0)),
            pl.BlockSpec((1, _TPB, _DIM), lambda b: (b, 0, 0)),
            pl.BlockSpec(memory_space=pltpu.SMEM),
            pl.BlockSpec(memory_space=pltpu.SMEM),
        ],
        out_shape=[
            jax.ShapeDtypeStruct((_N_TOK, _NUM_EMB), jnp.float32),
            jax.ShapeDtypeStruct((_NBLK, _TPB, _DIM), jnp.float32),
            jax.ShapeDtypeStruct((1, 1), jnp.float32),
            jax.ShapeDtypeStruct((1, 1), jnp.float32),
        ],
        scratch_shapes=[
            pltpu.VMEM((1, _NUM_EMB), jnp.float32),
            pltpu.SMEM((1, 1), jnp.float32),
            pltpu.VMEM((_DIM, _NUM_EMB), jnp.float32),
            pltpu.VMEM((1, _NUM_EMB), jnp.float32),
        ],
        compiler_params=pltpu.CompilerParams(
            dimension_semantics=("arbitrary",)),
    )(x_b, xn_b, xn2_b, weight)

    q_out = jnp.transpose(q_t.reshape(_B, _HW, _DIM), (0, 2, 1)).reshape(
        inputs.shape)
    return (q_out, loss.reshape(()), perp.reshape(()), enc)


# speedup vs baseline: 1.0384x; 1.0384x over previous
import jax
import jax.numpy as jnp
from jax.experimental import pallas as pl
from jax.experimental.pallas import tpu as pltpu

_NUM_EMB = 1024
_DIM = 64
_B = 16
_HW = 1024
_N_TOK = _B * _HW
_TPB = 1024
_NBLK = _N_TOK // _TPB
_EPS = 1e-12
_COMMIT = 0.25


def _rowsum64(a):
    t = a[:, 0:8]
    for i in range(1, 8):
        t = t + a[:, 8 * i:8 * i + 8]
    u = t[:, 0:4] + t[:, 4:8]
    v = u[:, 0:2] + u[:, 2:4]
    return v[:, 0:1] + v[:, 1:2]


def _vq_body(x_ref, xn_ref, xn2_ref, w_ref, enc_ref, q_ref, loss_ref,
             perp_ref, cnt_acc, loss_acc, wnt_s, wn2_s):
    b = pl.program_id(0)
    x = x_ref[0]
    xn = xn_ref[0]
    xn2 = xn2_ref[0]
    w = w_ref[...]

    @pl.when(b == 0)
    def _prep():
        wn = w / jnp.maximum(jnp.sqrt(_rowsum64(w * w)), _EPS)
        wnt_s[...] = -2.0 * wn.T
        wn2_s[...] = _rowsum64(wn * wn).T

    m2 = jax.lax.dot_general(
        xn, wnt_s[...], (((1,), (0,)), ((), ())),
        preferred_element_type=jnp.float32)
    d = (xn2 + wn2_s[...]) + m2

    iota_j = jax.lax.broadcasted_iota(jnp.int32, d.shape, 1)
    idx = jnp.argmin(d, axis=1).astype(jnp.int32)

    enc = (iota_j == idx[:, None]).astype(jnp.float32)
    enc_ref[...] = enc

    q = jax.lax.dot_general(
        enc, w, (((1,), (0,)), ((), ())),
        preferred_element_type=jnp.float32)
    q_ref[0] = x + (q - x)

    blk_sq = jnp.sum((q - x) ** 2)
    blk_cnt = jnp.sum(enc, axis=0)[None, :]

    @pl.when(b == 0)
    def _init():
        loss_acc[0, 0] = blk_sq
        cnt_acc[...] = blk_cnt

    @pl.when(b != 0)
    def _accum():
        loss_acc[0, 0] += blk_sq
        cnt_acc[...] += blk_cnt

    @pl.when(b == _NBLK - 1)
    def _finish():
        loss_ref[0, 0] = _COMMIT * (loss_acc[0, 0] / (_N_TOK * _DIM))
        avg = cnt_acc[...] * (1.0 / _N_TOK)
        perp_ref[0, 0] = jnp.exp(-jnp.sum(avg * jnp.log(avg + 1e-10)))


def kernel(inputs, weight, ema_w, ema_cluster_size):
    del ema_w, ema_cluster_size
    x4 = jnp.transpose(inputs, (0, 2, 3, 1))
    flat_x = x4.reshape(-1, _DIM)
    n = jnp.linalg.norm(flat_x, axis=1, keepdims=True)
    flat_x_n = flat_x / jnp.maximum(n, _EPS)
    xn2 = jnp.sum(flat_x_n ** 2, axis=1, keepdims=True)

    x_b = flat_x.reshape(_NBLK, _TPB, _DIM)
    xn_b = flat_x_n.reshape(_NBLK, _TPB, _DIM)
    xn2_b = xn2.reshape(_NBLK, _TPB, 1)

    enc, q_t, loss, perp = pl.pallas_call(
        _vq_body,
        grid=(_NBLK,),
        in_specs=[
            pl.BlockSpec((1, _TPB, _DIM), lambda b: (b, 0, 0)),
            pl.BlockSpec((1, _TPB, _DIM), lambda b: (b, 0, 0)),
            pl.BlockSpec((1, _TPB, 1), lambda b: (b, 0, 0)),
            pl.BlockSpec((_NUM_EMB, _DIM), lambda b: (0, 0)),
        ],
        out_specs=[
            pl.BlockSpec((_TPB, _NUM_EMB), lambda b: (b, 0)),
            pl.BlockSpec((1, _TPB, _DIM), lambda b: (b, 0, 0)),
            pl.BlockSpec(memory_space=pltpu.SMEM),
            pl.BlockSpec(memory_space=pltpu.SMEM),
        ],
        out_shape=[
            jax.ShapeDtypeStruct((_N_TOK, _NUM_EMB), jnp.float32),
            jax.ShapeDtypeStruct((_NBLK, _TPB, _DIM), jnp.float32),
            jax.ShapeDtypeStruct((1, 1), jnp.float32),
            jax.ShapeDtypeStruct((1, 1), jnp.float32),
        ],
        scratch_shapes=[
            pltpu.VMEM((1, _NUM_EMB), jnp.float32),
            pltpu.SMEM((1, 1), jnp.float32),
            pltpu.VMEM((_DIM, _NUM_EMB), jnp.float32),
            pltpu.VMEM((1, _NUM_EMB), jnp.float32),
        ],
        compiler_params=pltpu.CompilerParams(
            dimension_semantics=("arbitrary",)),
    )(x_b, xn_b, xn2_b, weight)

    q_out = jnp.transpose(q_t.reshape(_B, _HW, _DIM), (0, 2, 1)).reshape(
        inputs.shape)
    return (q_out, loss.reshape(()), perp.reshape(()), enc)
